# baseline (speedup 1.0000x reference)
import functools
import math

import jax
import jax.numpy as jnp
from jax import lax
from jax.experimental import pallas as pl
from jax.experimental.pallas import tpu as pltpu
from jax.experimental.pallas import tpu_sc as plsc

D_MODEL = 64
D_PAD = 128
SCALE = math.sqrt(D_MODEL)

NUM_CORES = 2
NUM_SUBCORES = 16
NUM_WORKERS = NUM_CORES * NUM_SUBCORES

SEQ = 200
SPLIT_A = 104
SPLIT_B = SEQ - SPLIT_A
NBUF = 2


def _emb_body(x_hbm, table_hbm, out_hbm, idx_v, *bufs_and_sems, sents_per_worker):
    gbufs = bufs_and_sems[:NBUF]
    sems = bufs_and_sems[NBUF : 2 * NBUF]
    obufs = bufs_and_sems[2 * NBUF : 2 * NBUF + 2]
    osems = bufs_and_sems[2 * NBUF + 2 : 2 * NBUF + 4]
    wid = lax.axis_index("s") * NUM_CORES + lax.axis_index("c")
    sent0 = wid * sents_per_worker
    idx0 = wid * (sents_per_worker * SEQ)

    def fire_gather(s_local, gbuf, sem):
        base = s_local * SEQ
        pltpu.async_copy(
            table_hbm.at[idx_v.at[pl.ds(base, SPLIT_A)]],
            gbuf.at[pl.ds(0, SPLIT_A)],
            sem,
        )
        pltpu.async_copy(
            table_hbm.at[idx_v.at[pl.ds(base + SPLIT_A, SPLIT_B)]],
            gbuf.at[pl.ds(SPLIT_A, SPLIT_B)],
            sem,
        )

    def wait_gather(s_local, gbuf, sem):
        base = s_local * SEQ
        pltpu.make_async_copy(
            table_hbm.at[idx_v.at[pl.ds(base, SPLIT_A)]],
            gbuf.at[pl.ds(0, SPLIT_A)],
            sem,
        ).wait()
        pltpu.make_async_copy(
            table_hbm.at[idx_v.at[pl.ds(base + SPLIT_A, SPLIT_B)]],
            gbuf.at[pl.ds(SPLIT_A, SPLIT_B)],
            sem,
        ).wait()

    def wait_out(s_local, obuf, osem):
        pltpu.make_async_copy(obuf, out_hbm.at[sent0 + s_local], osem).wait()

    pltpu.sync_copy(x_hbm.at[pl.ds(idx0, sents_per_worker * SEQ)], idx_v)

    fire_gather(0, gbufs[0], sems[0])

    def ring_body(p, carry):
        for r in range(NBUF):
            s = NBUF * p + r
            gbuf, sem = gbufs[r], sems[r]
            nbuf, nsem = gbufs[(r + 1) % NBUF], sems[(r + 1) % NBUF]
            obuf, osem = obufs[r % 2], osems[r % 2]

            @pl.when(s + 1 < sents_per_worker)
            def _():
                fire_gather(s + 1, nbuf, nsem)

            wait_gather(s, gbuf, sem)

            @pl.when(s >= 2)
            def _():
                wait_out(s - 2, obuf, osem)

            @plsc.parallel_loop(0, SEQ, unroll=8)
            def _crow(row):
                for c in range(D_MODEL // 16):
                    sl = pl.ds(c * 16, 16)
                    obuf[row, sl] = gbuf[row, sl]

            pltpu.async_copy(obuf, out_hbm.at[sent0 + s], osem)
        return carry

    lax.fori_loop(0, sents_per_worker // NBUF, ring_body, 0)

    wait_out(sents_per_worker - 2, obufs[0], osems[0])
    wait_out(sents_per_worker - 1, obufs[1], osems[1])


@jax.jit
def kernel(x, table):
    n_sent, seq = x.shape
    assert seq == SEQ and n_sent % NUM_WORKERS == 0
    sents_per_worker = n_sent // NUM_WORKERS

    table_wide = jnp.pad(
        table * jnp.float32(SCALE), ((0, 0), (0, D_PAD - D_MODEL))
    )
    x_flat = x.reshape(-1)

    mesh = plsc.VectorSubcoreMesh(core_axis_name="c", subcore_axis_name="s")
    out = pl.kernel(
        functools.partial(_emb_body, sents_per_worker=sents_per_worker),
        mesh=mesh,
        out_type=jax.ShapeDtypeStruct((n_sent, SEQ, D_MODEL), jnp.float32),
        scratch_types=[
            pltpu.VMEM((sents_per_worker * SEQ,), jnp.int32),
            *[pltpu.VMEM((SEQ, D_PAD), jnp.float32) for _ in range(NBUF)],
            *[pltpu.SemaphoreType.DMA for _ in range(NBUF)],
            pltpu.VMEM((SEQ, D_MODEL), jnp.float32),
            pltpu.VMEM((SEQ, D_MODEL), jnp.float32),
            pltpu.SemaphoreType.DMA,
            pltpu.SemaphoreType.DMA,
        ],
    )(x_flat, table_wide)
    return out


# speedup vs baseline: 1.2862x; 1.2862x over previous
import functools
import math

import jax
import jax.numpy as jnp
from jax import lax
from jax.experimental import pallas as pl
from jax.experimental.pallas import tpu as pltpu
from jax.experimental.pallas import tpu_sc as plsc

D_MODEL = 64
D_PAD = 128
SCALE = math.sqrt(D_MODEL)

NUM_CORES = 2
NUM_SUBCORES = 16
NUM_WORKERS = NUM_CORES * NUM_SUBCORES

SEQ = 200
SPLIT_A = 104
SPLIT_B = SEQ - SPLIT_A


def _emb_body(
    x_hbm,
    table_hbm,
    out_hbm,
    idx_v,
    g0,
    g1,
    o0,
    o1,
    gs0,
    gs1,
    os0,
    os1,
    *,
    sents_per_worker,
):
    wid = lax.axis_index("s") * NUM_CORES + lax.axis_index("c")
    sent0 = wid * sents_per_worker
    idx0 = wid * (sents_per_worker * SEQ)

    def fire_gather(s_local, gbuf, gsem):
        base = s_local * SEQ
        pltpu.async_copy(
            table_hbm.at[idx_v.at[pl.ds(base, SPLIT_A)]],
            gbuf.at[pl.ds(0, SPLIT_A)],
            gsem,
        )
        pltpu.async_copy(
            table_hbm.at[idx_v.at[pl.ds(base + SPLIT_A, SPLIT_B)]],
            gbuf.at[pl.ds(SPLIT_A, SPLIT_B)],
            gsem,
        )

    def wait_gather(s_local, gbuf, gsem):
        base = s_local * SEQ
        pltpu.make_async_copy(
            table_hbm.at[idx_v.at[pl.ds(base, SPLIT_A)]],
            gbuf.at[pl.ds(0, SPLIT_A)],
            gsem,
        ).wait()
        pltpu.make_async_copy(
            table_hbm.at[idx_v.at[pl.ds(base + SPLIT_A, SPLIT_B)]],
            gbuf.at[pl.ds(SPLIT_A, SPLIT_B)],
            gsem,
        ).wait()

    pltpu.sync_copy(x_hbm.at[pl.ds(idx0, sents_per_worker * SEQ)], idx_v)

    fire_gather(0, g0, gs0)

    def pair_body(p, carry):
        for b in (0, 1):
            s = 2 * p + b
            gbuf, gsem = (g0, gs0) if b == 0 else (g1, gs1)
            nbuf, nsem = (g1, gs1) if b == 0 else (g0, gs0)
            obuf, osem = (o0, os0) if b == 0 else (o1, os1)

            @pl.when(s + 1 < sents_per_worker)
            def _():
                fire_gather(s + 1, nbuf, nsem)

            wait_gather(s, gbuf, gsem)

            @pl.when(s >= 2)
            def _():
                pltpu.make_async_copy(obuf, out_hbm.at[sent0 + s - 2], osem).wait()

            def scale_row(r, c2):
                for c in range(D_MODEL // 16):
                    sl = pl.ds(c * 16, 16)
                    obuf[r, sl] = gbuf[r, sl] * SCALE
                return c2

            lax.fori_loop(0, SEQ, scale_row, 0)

            pltpu.async_copy(obuf, out_hbm.at[sent0 + s], osem)
        return carry

    lax.fori_loop(0, sents_per_worker // 2, pair_body, 0)

    pltpu.make_async_copy(
        o0, out_hbm.at[sent0 + sents_per_worker - 2], os0
    ).wait()
    pltpu.make_async_copy(
        o1, out_hbm.at[sent0 + sents_per_worker - 1], os1
    ).wait()


@jax.jit
def kernel(x, table):
    n_sent, seq = x.shape
    assert seq == SEQ and n_sent % NUM_WORKERS == 0
    sents_per_worker = n_sent // NUM_WORKERS

    table_wide = jnp.pad(table, ((0, 0), (0, D_PAD - D_MODEL)))
    x_flat = x.reshape(-1)

    mesh = plsc.VectorSubcoreMesh(core_axis_name="c", subcore_axis_name="s")
    out = pl.kernel(
        functools.partial(_emb_body, sents_per_worker=sents_per_worker),
        mesh=mesh,
        out_type=jax.ShapeDtypeStruct((n_sent, SEQ, D_MODEL), jnp.float32),
        scratch_types=[
            pltpu.VMEM((sents_per_worker * SEQ,), jnp.int32),
            pltpu.VMEM((SEQ, D_PAD), jnp.float32),
            pltpu.VMEM((SEQ, D_PAD), jnp.float32),
            pltpu.VMEM((SEQ, D_MODEL), jnp.float32),
            pltpu.VMEM((SEQ, D_MODEL), jnp.float32),
            pltpu.SemaphoreType.DMA,
            pltpu.SemaphoreType.DMA,
            pltpu.SemaphoreType.DMA,
            pltpu.SemaphoreType.DMA,
        ],
    )(x_flat, table_wide)
    return out
